# baseline (speedup 1.0000x reference)
import jax
import jax.numpy as jnp
from jax import lax
from jax.experimental import pallas as pl

_H = 64
_W = 64
_HW = _H * _W
_K = 16
_BB = 8
_R = _BB * _K


def _body(y_ref, r1_ref, r2_ref, fmb_ref, gt_ref, gf_ref):
    yb = y_ref[...]
    m = jnp.max(yb, axis=-1, keepdims=True)
    lane = lax.broadcasted_iota(jnp.int32, yb.shape, 1)
    cand = jnp.where(yb == m, lane, _HW)
    idx = jnp.min(cand, axis=-1, keepdims=True)
    idx = jnp.where(m > 0.0, idx, 0)
    px = idx & (_W - 1)
    py = idx >> 6
    c64 = lax.broadcasted_iota(jnp.int32, (_R, _W), 1)
    oh_y = (py == c64).astype(jnp.float32)
    oh_x = (px == c64).astype(jnp.float32)
    y1 = jnp.dot(oh_y, r1_ref[...], preferred_element_type=jnp.float32)
    x1 = jnp.dot(oh_x, r2_ref[...], preferred_element_type=jnp.float32)
    gt = y1 * x1
    gt_ref[...] = gt
    gf = jnp.dot(fmb_ref[...], gt, preferred_element_type=jnp.float32)
    gf_ref[...] = jnp.clip(gf, 0.0, 1.0)


def kernel(y, heatmaps, false_matrix):
    B, K, H, W = y.shape
    yf = y.reshape(B * K, H * W)
    g = heatmaps[:, 0, 0, :]
    r1 = jnp.repeat(g, W, axis=1)
    r2 = jnp.tile(g, (1, H))
    fmb = jnp.kron(jnp.eye(_BB, dtype=jnp.float32), false_matrix.T)
    grid = (B // _BB,)
    gt_flat, gf_flat = pl.pallas_call(
        _body,
        grid=grid,
        in_specs=[
            pl.BlockSpec((_R, _HW), lambda i: (i, 0)),
            pl.BlockSpec((_W, _HW), lambda i: (0, 0)),
            pl.BlockSpec((_W, _HW), lambda i: (0, 0)),
            pl.BlockSpec((_R, _R), lambda i: (0, 0)),
        ],
        out_specs=[
            pl.BlockSpec((_R, _HW), lambda i: (i, 0)),
            pl.BlockSpec((_R, _HW), lambda i: (i, 0)),
        ],
        out_shape=[
            jax.ShapeDtypeStruct((B * K, H * W), jnp.float32),
            jax.ShapeDtypeStruct((B * K, H * W), jnp.float32),
        ],
    )(yf, r1, r2, fmb)
    return gt_flat.reshape(B, K, H, W), gf_flat.reshape(B, K, H, W)


# speedup vs baseline: 1.1863x; 1.1863x over previous
import jax
import jax.numpy as jnp
from jax import lax
from jax.experimental import pallas as pl

_H = 64
_W = 64
_HW = _H * _W
_SIGMA = 2
_WIN = 3 * _SIGMA
_BB = 8


def _body(y_ref, fm_ref, gt_ref, gf_ref):
    yb = y_ref[...]
    m = jnp.max(yb, axis=(2, 3), keepdims=True)
    ioy = lax.broadcasted_iota(jnp.int32, yb.shape, 2)
    iox = lax.broadcasted_iota(jnp.int32, yb.shape, 3)
    flat = ioy * _W + iox
    cand = jnp.where(yb == m, flat, _HW)
    idx = jnp.min(cand, axis=(2, 3), keepdims=True)
    idx = jnp.where(m > 0.0, idx, 0)
    px = idx & (_W - 1)
    py = idx >> 6
    dx = iox - px
    dy = ioy - py
    d2 = (dx * dx + dy * dy).astype(jnp.float32)
    inwin = (jnp.abs(dx) <= _WIN) & (jnp.abs(dy) <= _WIN)
    gt = jnp.where(inwin, jnp.exp(d2 * (-1.0 / (2.0 * _SIGMA * _SIGMA))), 0.0)
    gt_ref[...] = gt
    fm = fm_ref[...]
    a_col = (jnp.sum(fm, axis=1, keepdims=True) > 0.0).astype(jnp.float32)
    a_b = a_col[None, :, :, None]
    t = jnp.sum(gt * a_b, axis=1, keepdims=True)
    gf_ref[...] = jnp.clip(t - gt, 0.0, 1.0) * a_b


def kernel(y, heatmaps, false_matrix):
    B, K, H, W = y.shape
    grid = (B // _BB,)
    gt, gf = pl.pallas_call(
        _body,
        grid=grid,
        in_specs=[
            pl.BlockSpec((_BB, K, H, W), lambda i: (i, 0, 0, 0)),
            pl.BlockSpec((K, K), lambda i: (0, 0)),
        ],
        out_specs=[
            pl.BlockSpec((_BB, K, H, W), lambda i: (i, 0, 0, 0)),
            pl.BlockSpec((_BB, K, H, W), lambda i: (i, 0, 0, 0)),
        ],
        out_shape=[
            jax.ShapeDtypeStruct((B, K, H, W), jnp.float32),
            jax.ShapeDtypeStruct((B, K, H, W), jnp.float32),
        ],
    )(y, false_matrix)
    return gt, gf


# speedup vs baseline: 1.9370x; 1.6328x over previous
import jax
import jax.numpy as jnp
from jax import lax
from jax.experimental import pallas as pl

_H = 64
_W = 64
_HW = _H * _W
_K = 16
_BB = 8


def _body(y_ref, r1_ref, r2_ref, fmt_ref, gt_ref, gf_ref):
    yb = y_ref[...]
    m = jnp.max(yb, axis=-1, keepdims=True)
    lane = lax.broadcasted_iota(jnp.int32, yb.shape, 2)
    cand = jnp.where(yb == m, lane, _HW)
    idx = jnp.min(cand, axis=-1, keepdims=True)
    idx = jnp.where(m > 0.0, idx, 0)
    px = idx & (_W - 1)
    py = idx >> 6
    c64 = lax.broadcasted_iota(jnp.int32, (_BB, _K, _W), 2)
    oh_y = (py == c64).astype(jnp.float32)
    oh_x = (px == c64).astype(jnp.float32)
    r1 = r1_ref[...]
    r2 = r2_ref[...]
    fmt = fmt_ref[...]
    for b in range(_BB):
        y1 = jnp.dot(oh_y[b], r1, preferred_element_type=jnp.float32)
        x1 = jnp.dot(oh_x[b], r2, preferred_element_type=jnp.float32)
        gt_b = y1 * x1
        gt_ref[b, :, :] = gt_b
        gf_b = jnp.dot(fmt, gt_b, preferred_element_type=jnp.float32)
        gf_ref[b, :, :] = jnp.clip(gf_b, 0.0, 1.0)


def kernel(y, heatmaps, false_matrix):
    B, K, H, W = y.shape
    yf = y.reshape(B, K, H * W)
    g = heatmaps[:, 0, 0, :]
    r1 = jnp.repeat(g, W, axis=1)
    r2 = jnp.tile(g, (1, H))
    fmt = false_matrix.T
    grid = (B // _BB,)
    gt, gf = pl.pallas_call(
        _body,
        grid=grid,
        in_specs=[
            pl.BlockSpec((_BB, K, H * W), lambda i: (i, 0, 0)),
            pl.BlockSpec((_W, _HW), lambda i: (0, 0)),
            pl.BlockSpec((_W, _HW), lambda i: (0, 0)),
            pl.BlockSpec((K, K), lambda i: (0, 0)),
        ],
        out_specs=[
            pl.BlockSpec((_BB, K, H * W), lambda i: (i, 0, 0)),
            pl.BlockSpec((_BB, K, H * W), lambda i: (i, 0, 0)),
        ],
        out_shape=[
            jax.ShapeDtypeStruct((B, K, H * W), jnp.float32),
            jax.ShapeDtypeStruct((B, K, H * W), jnp.float32),
        ],
    )(yf, r1, r2, fmt)
    return gt.reshape(B, K, H, W), gf.reshape(B, K, H, W)


# speedup vs baseline: 2.0509x; 1.0588x over previous
import jax
import jax.numpy as jnp
from jax import lax
from jax.experimental import pallas as pl

_H = 64
_W = 64
_HW = _H * _W
_K = 16
_BB = 8


def _body(y_ref, r1_ref, r2_ref, fmb_ref, gt_ref, gf_ref):
    yb = y_ref[...]
    m = jnp.max(yb, axis=-1, keepdims=True)
    lane = lax.broadcasted_iota(jnp.int32, yb.shape, 2)
    cand = jnp.where(yb == m, lane, _HW)
    idx = jnp.min(cand, axis=-1, keepdims=True)
    idx = jnp.where(m > 0.0, idx, 0)
    idx2 = jnp.reshape(idx, (_BB * _K, 1))
    px = idx2 & (_W - 1)
    py = idx2 >> 6
    c64 = lax.broadcasted_iota(jnp.int32, (_BB * _K, _W), 1)
    oh_y = (py == c64).astype(jnp.float32)
    oh_x = (px == c64).astype(jnp.float32)
    y1 = jnp.dot(oh_y, r1_ref[...], preferred_element_type=jnp.float32)
    x1 = jnp.dot(oh_x, r2_ref[...], preferred_element_type=jnp.float32)
    gt = y1 * x1
    gf = jnp.dot(fmb_ref[...], gt, preferred_element_type=jnp.float32)
    gf = jnp.clip(gf, 0.0, 1.0)
    for b in range(_BB):
        gt_ref[b, :, :] = gt[b * _K:(b + 1) * _K, :]
        gf_ref[b, :, :] = gf[b * _K:(b + 1) * _K, :]


def kernel(y, heatmaps, false_matrix):
    B, K, H, W = y.shape
    yf = y.reshape(B, K, H * W)
    g = heatmaps[:, 0, 0, :]
    r1 = jnp.repeat(g, W, axis=1)
    r2 = jnp.tile(g, (1, H))
    fmb = jnp.kron(jnp.eye(_BB, dtype=jnp.float32), false_matrix.T)
    grid = (B // _BB,)
    gt, gf = pl.pallas_call(
        _body,
        grid=grid,
        in_specs=[
            pl.BlockSpec((_BB, K, H * W), lambda i: (i, 0, 0)),
            pl.BlockSpec((_W, _HW), lambda i: (0, 0)),
            pl.BlockSpec((_W, _HW), lambda i: (0, 0)),
            pl.BlockSpec((_BB * _K, _BB * _K), lambda i: (0, 0)),
        ],
        out_specs=[
            pl.BlockSpec((_BB, K, H * W), lambda i: (i, 0, 0)),
            pl.BlockSpec((_BB, K, H * W), lambda i: (i, 0, 0)),
        ],
        out_shape=[
            jax.ShapeDtypeStruct((B, K, H * W), jnp.float32),
            jax.ShapeDtypeStruct((B, K, H * W), jnp.float32),
        ],
    )(yf, r1, r2, fmb)
    return gt.reshape(B, K, H, W), gf.reshape(B, K, H, W)


# speedup vs baseline: 2.1381x; 1.0425x over previous
import jax
import jax.numpy as jnp
from jax import lax
from jax.experimental import pallas as pl

_H = 64
_W = 64
_HW = _H * _W
_K = 16
_BB = 16


def _body(y_ref, r1_ref, r2_ref, fmb_ref, gt_ref, gf_ref):
    yb = y_ref[...]
    m = jnp.max(yb, axis=-1, keepdims=True)
    lane = lax.broadcasted_iota(jnp.int32, yb.shape, 2)
    cand = jnp.where(yb == m, lane, _HW)
    idx = jnp.min(cand, axis=-1, keepdims=True)
    idx = jnp.where(m > 0.0, idx, 0)
    idx2 = jnp.reshape(idx, (_BB * _K, 1))
    px = idx2 & (_W - 1)
    py = idx2 >> 6
    c64 = lax.broadcasted_iota(jnp.int32, (_BB * _K, _W), 1)
    oh_y = (py == c64).astype(jnp.float32)
    oh_x = (px == c64).astype(jnp.float32)
    y1 = jnp.dot(oh_y, r1_ref[...], preferred_element_type=jnp.float32)
    x1 = jnp.dot(oh_x, r2_ref[...], preferred_element_type=jnp.float32)
    gt = y1 * x1
    gf = jnp.dot(fmb_ref[...], gt, preferred_element_type=jnp.float32)
    gf = jnp.clip(gf, 0.0, 1.0)
    for b in range(_BB):
        gt_ref[b, :, :] = gt[b * _K:(b + 1) * _K, :]
        gf_ref[b, :, :] = gf[b * _K:(b + 1) * _K, :]


def kernel(y, heatmaps, false_matrix):
    B, K, H, W = y.shape
    yf = y.reshape(B, K, H * W)
    g = heatmaps[:, 0, 0, :]
    r1 = jnp.repeat(g, W, axis=1)
    r2 = jnp.tile(g, (1, H))
    fmb = jnp.kron(jnp.eye(_BB, dtype=jnp.float32), false_matrix.T)
    grid = (B // _BB,)
    gt, gf = pl.pallas_call(
        _body,
        grid=grid,
        in_specs=[
            pl.BlockSpec((_BB, K, H * W), lambda i: (i, 0, 0)),
            pl.BlockSpec((_W, _HW), lambda i: (0, 0)),
            pl.BlockSpec((_W, _HW), lambda i: (0, 0)),
            pl.BlockSpec((_BB * _K, _BB * _K), lambda i: (0, 0)),
        ],
        out_specs=[
            pl.BlockSpec((_BB, K, H * W), lambda i: (i, 0, 0)),
            pl.BlockSpec((_BB, K, H * W), lambda i: (i, 0, 0)),
        ],
        out_shape=[
            jax.ShapeDtypeStruct((B, K, H * W), jnp.float32),
            jax.ShapeDtypeStruct((B, K, H * W), jnp.float32),
        ],
    )(yf, r1, r2, fmb)
    return gt.reshape(B, K, H, W), gf.reshape(B, K, H, W)
